# baseline (speedup 1.0000x reference)
import functools

import jax
import jax.numpy as jnp
from jax import lax
from jax.experimental import pallas as pl
from jax.experimental.pallas import tpu as pltpu
from jax.experimental.pallas import tpu_sc as plsc

_B, _K, _N, _D = 4096, 8, 1024, 256
_BB = 512


def _argmin_tc_body(x_ref, e_ref, idx_ref, gidx_ref, e2_ref):
    @pl.when(pl.program_id(0) == 0)
    def _():
        e2_ref[...] = 0.5 * jnp.sum(e_ref[...] * e_ref[...], axis=-1)

    lane_k = lax.broadcasted_iota(jnp.int32, (_BB, _K), 1)
    acc = jnp.zeros((_BB, _K), jnp.int32)
    for k in range(_K):
        xk = x_ref[:, k, :]
        ek = e_ref[k]
        cross = lax.dot_general(xk, ek, (((1,), (1,)), ((), ())),
                                preferred_element_type=jnp.float32)
        score = e2_ref[k:k + 1, :] - cross
        idxk = jnp.argmin(score, axis=1, keepdims=True).astype(jnp.int32)
        acc = jnp.where(lane_k == k, idxk, acc)
    idx_ref[...] = acc
    gidx_ref[...] = acc + lane_k * _N


def _argmin_call(x, entries, chunk, nsteps):
    bc = nsteps * _BB
    return pl.pallas_call(
        _argmin_tc_body,
        grid=(nsteps,),
        in_specs=[
            pl.BlockSpec((_BB, _K, _D), lambda i: (chunk * nsteps + i, 0, 0)),
            pl.BlockSpec((_K, _N, _D), lambda i: (0, 0, 0)),
        ],
        out_specs=[
            pl.BlockSpec((_BB, _K), lambda i: (i, 0)),
            pl.BlockSpec((_BB, _K), lambda i: (i, 0)),
        ],
        out_shape=[
            jax.ShapeDtypeStruct((bc, _K), jnp.int32),
            jax.ShapeDtypeStruct((bc, _K), jnp.int32),
        ],
        scratch_shapes=[pltpu.VMEM((_K, _N), jnp.float32)],
    )(x, entries)


_ROWS = _B * _K
_CH = 128


_NB = 3


def _gather_sc_body(nc, rpw, table_hbm, gidx_hbm, out_hbm,
                    idx_v, buf0, buf1, buf2, gs0, gs1, gs2, ss0, ss1, ss2):
    wid = lax.axis_index("s") * nc + lax.axis_index("c")
    base = wid * rpw
    pltpu.sync_copy(gidx_hbm.at[pl.ds(base, rpw)], idx_v)
    bufs = (buf0, buf1, buf2)
    gs = (gs0, gs1, gs2)
    ss = (ss0, ss1, ss2)
    nch = rpw // _CH

    def gather(c):
        return pltpu.async_copy(
            table_hbm.at[idx_v.at[pl.ds(c * _CH, _CH)]],
            bufs[c % _NB], gs[c % _NB])

    g = [None] * nch
    s = [None] * nch
    for c in range(_NB - 1):
        g[c] = gather(c)
    for c in range(nch):
        f = c + _NB - 1
        if f < nch:
            if c >= 1:
                s[c - 1].wait()
            g[f] = gather(f)
        g[c].wait()
        s[c] = pltpu.async_copy(
            bufs[c % _NB], out_hbm.at[pl.ds(base + c * _CH, _CH)], ss[c % _NB])
    for c in range(max(0, nch - _NB), nch):
        s[c].wait()


def _gather_call(table, gidx_flat):
    info = plsc.get_sparse_core_info()
    nw = info.num_cores * info.num_subcores
    nrows = gidx_flat.shape[0]
    rpw = nrows // nw
    fn = pl.kernel(
        functools.partial(_gather_sc_body, info.num_cores, rpw),
        out_type=jax.ShapeDtypeStruct((nrows, _D), jnp.float32),
        mesh=plsc.VectorSubcoreMesh(core_axis_name="c", subcore_axis_name="s"),
        scratch_types=[
            pltpu.VMEM((rpw,), jnp.int32),
            pltpu.VMEM((_CH, _D), jnp.float32),
            pltpu.VMEM((_CH, _D), jnp.float32),
            pltpu.VMEM((_CH, _D), jnp.float32),
            pltpu.SemaphoreType.DMA,
            pltpu.SemaphoreType.DMA,
            pltpu.SemaphoreType.DMA,
            pltpu.SemaphoreType.DMA,
            pltpu.SemaphoreType.DMA,
            pltpu.SemaphoreType.DMA,
        ],
    )
    return fn(table, gidx_flat)


_NCK = 4


def kernel(x, entries):
    table = entries.reshape(_K * _N, _D)
    nsteps = _B // (_NCK * _BB)
    bc = nsteps * _BB
    idxs, qs = [], []
    for j in range(_NCK):
        idx_j, gidx_j = _argmin_call(x, entries, j, nsteps)
        qs.append(_gather_call(table, gidx_j.reshape(bc * _K)))
        idxs.append(idx_j)
    idx = jnp.concatenate(idxs, axis=0)
    q = jnp.concatenate(qs, axis=0).reshape(_B, _K, _D)
    return q, idx


# speedup vs baseline: 1.2076x; 1.2076x over previous
import functools

import jax
import jax.numpy as jnp
from jax import lax
from jax.experimental import pallas as pl
from jax.experimental.pallas import tpu as pltpu
from jax.experimental.pallas import tpu_sc as plsc

_B, _K, _N, _D = 4096, 8, 1024, 256
_BB = 512


def _argmin_tc_body(x_ref, e_ref, idx_ref, gidx_ref, e2_ref):
    @pl.when(pl.program_id(0) == 0)
    def _():
        e2_ref[...] = 0.5 * jnp.sum(e_ref[...] * e_ref[...], axis=-1)

    lane_k = lax.broadcasted_iota(jnp.int32, (_BB, _K), 1)
    acc = jnp.zeros((_BB, _K), jnp.int32)
    for k in range(_K):
        xk = x_ref[:, k, :]
        ek = e_ref[k]
        cross = lax.dot_general(xk, ek, (((1,), (1,)), ((), ())),
                                preferred_element_type=jnp.float32)
        score = e2_ref[k:k + 1, :] - cross
        idxk = jnp.argmin(score, axis=1, keepdims=True).astype(jnp.int32)
        acc = jnp.where(lane_k == k, idxk, acc)
    idx_ref[...] = acc
    gidx_ref[...] = acc + lane_k * _N


def _argmin_call(x, entries, chunk, nsteps):
    bc = nsteps * _BB
    return pl.pallas_call(
        _argmin_tc_body,
        grid=(nsteps,),
        in_specs=[
            pl.BlockSpec((_BB, _K, _D), lambda i: (chunk * nsteps + i, 0, 0)),
            pl.BlockSpec((_K, _N, _D), lambda i: (0, 0, 0)),
        ],
        out_specs=[
            pl.BlockSpec((_BB, _K), lambda i: (i, 0)),
            pl.BlockSpec((_BB, _K), lambda i: (i, 0)),
        ],
        out_shape=[
            jax.ShapeDtypeStruct((bc, _K), jnp.int32),
            jax.ShapeDtypeStruct((bc, _K), jnp.int32),
        ],
        scratch_shapes=[pltpu.VMEM((_K, _N), jnp.float32)],
    )(x, entries)


_ROWS = _B * _K
_CH = 128


_NB = 3


def _gather_sc_body(nc, rpw, table_hbm, gidx_hbm, out_hbm,
                    idx_v, buf0, buf1, buf2, gs0, gs1, gs2, ss0, ss1, ss2):
    wid = lax.axis_index("s") * nc + lax.axis_index("c")
    base = wid * rpw
    pltpu.sync_copy(gidx_hbm.at[pl.ds(base, rpw)], idx_v)
    bufs = (buf0, buf1, buf2)
    gs = (gs0, gs1, gs2)
    ss = (ss0, ss1, ss2)
    nch = rpw // _CH

    def gather(c):
        return pltpu.async_copy(
            table_hbm.at[idx_v.at[pl.ds(c * _CH, _CH)]],
            bufs[c % _NB], gs[c % _NB])

    g = [None] * nch
    s = [None] * nch
    for c in range(_NB - 1):
        g[c] = gather(c)
    for c in range(nch):
        f = c + _NB - 1
        if f < nch:
            if c >= 1:
                s[c - 1].wait()
            g[f] = gather(f)
        g[c].wait()
        s[c] = pltpu.async_copy(
            bufs[c % _NB], out_hbm.at[pl.ds(base + c * _CH, _CH)], ss[c % _NB])
    for c in range(max(0, nch - _NB), nch):
        s[c].wait()


def _gather_call(table, gidx_flat, out_rows=None):
    info = plsc.get_sparse_core_info()
    nw = info.num_cores * info.num_subcores
    nrows = gidx_flat.shape[0]
    rpw = nrows // nw
    fn = pl.kernel(
        functools.partial(_gather_sc_body, info.num_cores, rpw),
        out_type=jax.ShapeDtypeStruct((out_rows or nrows, _D), jnp.float32),
        mesh=plsc.VectorSubcoreMesh(core_axis_name="c", subcore_axis_name="s"),
        scratch_types=[
            pltpu.VMEM((rpw,), jnp.int32),
            pltpu.VMEM((_CH, _D), jnp.float32),
            pltpu.VMEM((_CH, _D), jnp.float32),
            pltpu.VMEM((_CH, _D), jnp.float32),
            pltpu.SemaphoreType.DMA,
            pltpu.SemaphoreType.DMA,
            pltpu.SemaphoreType.DMA,
            pltpu.SemaphoreType.DMA,
            pltpu.SemaphoreType.DMA,
            pltpu.SemaphoreType.DMA,
        ],
    )
    return fn(table, gidx_flat)


_NCK = 2


def kernel(x, entries):
    table = entries.reshape(_K * _N, _D)
    nsteps = _B // (_NCK * _BB)
    bc = nsteps * _BB
    idxs, qs = [], []
    for j in range(_NCK):
        idx_j, gidx_j = _argmin_call(x, entries, j, nsteps)
        qs.append(_gather_call(table, gidx_j.reshape(bc * _K),
                               out_rows=_ROWS if j == 0 else None))
        idxs.append(idx_j)
    q = qs[0]
    for j in range(1, _NCK):
        q = lax.dynamic_update_slice(q, qs[j], (j * bc * _K, 0))
    idx = jnp.concatenate(idxs, axis=0)
    return q.reshape(_B, _K, _D), idx


# speedup vs baseline: 1.2285x; 1.0173x over previous
import functools

import jax
import jax.numpy as jnp
from jax import lax
from jax.experimental import pallas as pl
from jax.experimental.pallas import tpu as pltpu
from jax.experimental.pallas import tpu_sc as plsc

_B, _K, _N, _D = 4096, 8, 1024, 256
_BB = 512


def _argmin_tc_body(x_ref, e_ref, idx_ref, gidx_ref, e2_ref):
    @pl.when(pl.program_id(0) == 0)
    def _():
        e2_ref[...] = 0.5 * jnp.sum(e_ref[...] * e_ref[...], axis=-1)

    lane_k = lax.broadcasted_iota(jnp.int32, (_BB, _K), 1)
    acc = jnp.zeros((_BB, _K), jnp.int32)
    for k in range(_K):
        xk = x_ref[:, k, :]
        ek = e_ref[k]
        cross = lax.dot_general(xk, ek, (((1,), (1,)), ((), ())),
                                preferred_element_type=jnp.float32)
        score = e2_ref[k:k + 1, :] - cross
        idxk = jnp.argmin(score, axis=1, keepdims=True).astype(jnp.int32)
        acc = jnp.where(lane_k == k, idxk, acc)
    idx_ref[...] = acc
    gidx_ref[...] = acc + lane_k * _N


def _argmin_call(x, entries, chunk, nsteps):
    bc = nsteps * _BB
    return pl.pallas_call(
        _argmin_tc_body,
        grid=(nsteps,),
        in_specs=[
            pl.BlockSpec((_BB, _K, _D), lambda i: (chunk * nsteps + i, 0, 0)),
            pl.BlockSpec((_K, _N, _D), lambda i: (0, 0, 0)),
        ],
        out_specs=[
            pl.BlockSpec((_BB, _K), lambda i: (i, 0)),
            pl.BlockSpec((_BB, _K), lambda i: (i, 0)),
        ],
        out_shape=[
            jax.ShapeDtypeStruct((bc, _K), jnp.int32),
            jax.ShapeDtypeStruct((bc, _K), jnp.int32),
        ],
        scratch_shapes=[pltpu.VMEM((_K, _N), jnp.float32)],
    )(x, entries)


_ROWS = _B * _K
_CH = 64
_NB = 6
_PF = 3


def _gather_sc_body(nc, rpw, table_hbm, gidx_hbm, out_hbm, idx_v, *scr):
    bufs = scr[:_NB]
    gs = scr[_NB:2 * _NB]
    ss = scr[2 * _NB:3 * _NB]
    wid = lax.axis_index("s") * nc + lax.axis_index("c")
    base = wid * rpw
    pltpu.sync_copy(gidx_hbm.at[pl.ds(base, rpw)], idx_v)
    nch = rpw // _CH

    def gather(c):
        return pltpu.async_copy(
            table_hbm.at[idx_v.at[pl.ds(c * _CH, _CH)]],
            bufs[c % _NB], gs[c % _NB])

    g = [None] * nch
    s = [None] * nch
    for c in range(min(_PF, nch)):
        g[c] = gather(c)
    for c in range(nch):
        f = c + _PF
        if f < nch:
            w = f - _NB
            if w >= 0:
                s[w].wait()
            g[f] = gather(f)
        g[c].wait()
        s[c] = pltpu.async_copy(
            bufs[c % _NB], out_hbm.at[pl.ds(base + c * _CH, _CH)], ss[c % _NB])
    for c in range(max(0, nch - _NB), nch):
        s[c].wait()


def _gather_call(table, gidx_flat, out_rows=None):
    info = plsc.get_sparse_core_info()
    nw = info.num_cores * info.num_subcores
    nrows = gidx_flat.shape[0]
    rpw = nrows // nw
    fn = pl.kernel(
        functools.partial(_gather_sc_body, info.num_cores, rpw),
        out_type=jax.ShapeDtypeStruct((out_rows or nrows, _D), jnp.float32),
        mesh=plsc.VectorSubcoreMesh(core_axis_name="c", subcore_axis_name="s"),
        scratch_types=(
            [pltpu.VMEM((rpw,), jnp.int32)]
            + [pltpu.VMEM((_CH, _D), jnp.float32) for _ in range(_NB)]
            + [pltpu.SemaphoreType.DMA for _ in range(2 * _NB)]
        ),
    )
    return fn(table, gidx_flat)


_NCK = 1


def kernel(x, entries):
    table = entries.reshape(_K * _N, _D)
    nsteps = _B // (_NCK * _BB)
    bc = nsteps * _BB
    idxs, qs = [], []
    for j in range(_NCK):
        idx_j, gidx_j = _argmin_call(x, entries, j, nsteps)
        qs.append(_gather_call(table, gidx_j.reshape(bc * _K),
                               out_rows=_ROWS if j == 0 else None))
        idxs.append(idx_j)
    q = qs[0]
    for j in range(1, _NCK):
        q = lax.dynamic_update_slice(q, qs[j], (j * bc * _K, 0))
    idx = jnp.concatenate(idxs, axis=0)
    return q.reshape(_B, _K, _D), idx
